# baseline (speedup 1.0000x reference)
import functools

import jax
import jax.numpy as jnp
from jax import lax
from jax.experimental import pallas as pl
from jax.experimental.pallas import tpu as pltpu
from jax.experimental.pallas import tpu_sc as plsc

NC = 2
NS = 16
N = 10000
NPAD = 10240


def _mesh():
    return plsc.VectorSubcoreMesh(
        core_axis_name="c", subcore_axis_name="s", num_cores=NC, num_subcores=NS
    )


_SC_PARAMS = pltpu.CompilerParams(use_tc_tiling_on_sc=False, needs_layout_passes=False)


def _sc_degree(e, b):
    et = e // (NC * NS)
    rt = NPAD // NS

    n = et // b

    @functools.partial(
        pl.kernel,
        out_type=jax.ShapeDtypeStruct((NC, NPAD, 1), jnp.float32),
        mesh=_mesh(),
        compiler_params=_SC_PARAMS,
        scratch_types=[
            pltpu.VMEM((b,), jnp.int32),
            pltpu.VMEM((b,), jnp.int32),
            pltpu.VMEM((b, 1), jnp.float32),
            pltpu.VMEM_SHARED((NPAD, 1), jnp.float32),
            pltpu.SemaphoreType.DMA,
            pltpu.SemaphoreType.DMA,
        ],
    )
    def k(dst_hbm, zeros_hbm, ones_hbm, out_hbm, di0, di1, ones_v, acc_sh, smi0, smi1):
        didx = [di0, di1]
        smi = [smi0, smi1]
        c = lax.axis_index("c")
        s = lax.axis_index("s")
        rbase = s * rt
        ebase = (c * NS + s) * et
        dsc = [None] * n

        def issue_idx(j):
            k2 = j % 2
            dsc[j] = pltpu.async_copy(dst_hbm.at[pl.ds(ebase + j * b, b)], didx[k2], smi[k2])

        issue_idx(0)
        if n > 1:
            issue_idx(1)
        pltpu.sync_copy(zeros_hbm.at[pl.ds(rbase, rt)], acc_sh.at[pl.ds(rbase, rt)])
        pltpu.sync_copy(ones_hbm, ones_v)
        plsc.subcore_barrier()
        for j in range(n):
            dsc[j].wait()
            pltpu.sync_copy(ones_v, acc_sh.at[didx[j % 2]], add=True)
            if j + 2 < n:
                issue_idx(j + 2)
        plsc.subcore_barrier()
        pltpu.sync_copy(acc_sh.at[pl.ds(rbase, rt)], out_hbm.at[c].at[pl.ds(rbase, rt)])

    return k


def _sc_scatter_rows(d, e, b):
    ec = e // NC
    et = ec // NS
    rt = NPAD // NS

    n = et // b

    @functools.partial(
        pl.kernel,
        out_type=jax.ShapeDtypeStruct((NC, NPAD, d), jnp.float32),
        mesh=_mesh(),
        compiler_params=_SC_PARAMS,
        scratch_types=[
            pltpu.VMEM((b,), jnp.int32),
            pltpu.VMEM((b,), jnp.int32),
            pltpu.VMEM((b,), jnp.int32),
            pltpu.VMEM((b,), jnp.int32),
            pltpu.VMEM((b, d), jnp.float32),
            pltpu.VMEM((b, d), jnp.float32),
            pltpu.VMEM_SHARED((NPAD, d), jnp.float32),
            pltpu.SemaphoreType.DMA,
            pltpu.SemaphoreType.DMA,
            pltpu.SemaphoreType.DMA,
            pltpu.SemaphoreType.DMA,
        ],
    )
    def k(t_hbm, src_hbm, dst_hbm, zeros_hbm, out_hbm,
          si0, si1, di0, di1, r0, r1, acc_sh, smi0, smi1, smr0, smr1):
        sidx = [si0, si1]
        didx = [di0, di1]
        rows = [r0, r1]
        smi = [smi0, smi1]
        smr = [smr0, smr1]
        c = lax.axis_index("c")
        s = lax.axis_index("s")
        rbase = s * rt
        ebase = c * ec + s * et
        dsc_i = [None] * n
        dsc_d = [None] * n
        dsc_r = [None] * n

        def issue_idx(j):
            k2 = j % 2
            off = ebase + j * b
            dsc_i[j] = pltpu.async_copy(src_hbm.at[pl.ds(off, b)], sidx[k2], smi[k2])
            dsc_d[j] = pltpu.async_copy(dst_hbm.at[pl.ds(off, b)], didx[k2], smi[k2])

        def issue_gather(j):
            k2 = j % 2
            dsc_i[j].wait()
            dsc_d[j].wait()
            dsc_r[j] = pltpu.async_copy(t_hbm.at[sidx[k2]], rows[k2], smr[k2])

        issue_idx(0)
        issue_gather(0)
        if n > 1:
            issue_idx(1)
        pltpu.sync_copy(zeros_hbm.at[pl.ds(rbase, rt)], acc_sh.at[pl.ds(rbase, rt)])
        plsc.subcore_barrier()
        for j in range(n):
            if j + 1 < n:
                issue_gather(j + 1)
            dsc_r[j].wait()
            pltpu.sync_copy(rows[j % 2], acc_sh.at[didx[j % 2]], add=True)
            if j + 2 < n:
                issue_idx(j + 2)
        plsc.subcore_barrier()
        pltpu.sync_copy(acc_sh.at[pl.ds(rbase, rt)], out_hbm.at[c].at[pl.ds(rbase, rt)])

    return k


def _sc_edge_dot(e_pad, b):
    et = e_pad // (NC * NS)

    n = et // b

    @functools.partial(
        pl.kernel,
        out_type=jax.ShapeDtypeStruct((e_pad,), jnp.float32),
        mesh=_mesh(),
        compiler_params=_SC_PARAMS,
        scratch_types=[
            pltpu.VMEM((b,), jnp.int32),
            pltpu.VMEM((b,), jnp.int32),
            pltpu.VMEM((b,), jnp.int32),
            pltpu.VMEM((b,), jnp.int32),
            pltpu.VMEM((b, 16), jnp.float32),
            pltpu.VMEM((b, 16), jnp.float32),
            pltpu.VMEM((b, 16), jnp.float32),
            pltpu.VMEM((b, 16), jnp.float32),
            pltpu.VMEM((b,), jnp.float32),
            pltpu.VMEM((b,), jnp.float32),
            pltpu.SemaphoreType.DMA,
            pltpu.SemaphoreType.DMA,
            pltpu.SemaphoreType.DMA,
            pltpu.SemaphoreType.DMA,
            pltpu.SemaphoreType.DMA,
            pltpu.SemaphoreType.DMA,
        ],
    )
    def k(h_hbm, ii_hbm, jj_hbm, out_hbm,
          ia0, ia1, ib0, ib1, ra0, ra1, rb0, rb1, ac0, ac1,
          smi0, smi1, smr0, smr1, smw0, smw1):
        ia = [ia0, ia1]
        ib = [ib0, ib1]
        ra = [ra0, ra1]
        rb = [rb0, rb1]
        acc_v = [ac0, ac1]
        smi = [smi0, smi1]
        smr = [smr0, smr1]
        smw = [smw0, smw1]
        wid = lax.axis_index("c") * NS + lax.axis_index("s")
        base = wid * et
        lanes = lax.iota(jnp.int32, 16)
        dsc_ia = [None] * n
        dsc_ib = [None] * n
        dsc_ra = [None] * n
        dsc_rb = [None] * n
        dsc_w = [None] * n

        def issue_idx(j):
            k2 = j % 2
            off = base + j * b
            dsc_ia[j] = pltpu.async_copy(ii_hbm.at[pl.ds(off, b)], ia[k2], smi[k2])
            dsc_ib[j] = pltpu.async_copy(jj_hbm.at[pl.ds(off, b)], ib[k2], smi[k2])

        def issue_gather(j):
            k2 = j % 2
            dsc_ia[j].wait()
            dsc_ib[j].wait()
            dsc_ra[j] = pltpu.async_copy(h_hbm.at[ia[k2]], ra[k2], smr[k2])
            dsc_rb[j] = pltpu.async_copy(h_hbm.at[ib[k2]], rb[k2], smr[k2])

        issue_idx(0)
        issue_gather(0)
        if n > 1:
            issue_idx(1)
        for j in range(n):
            k2 = j % 2
            if j + 1 < n:
                issue_gather(j + 1)
            if j >= 2:
                dsc_w[j - 2].wait()
            dsc_ra[j].wait()
            dsc_rb[j].wait()
            if j + 2 < n:
                issue_idx(j + 2)

            def grp(g, carry2, _k2=k2):
                ridx = g * 16 + lanes
                acc = jnp.zeros((16,), jnp.float32)
                for f in range(16):
                    cidx = jnp.full((16,), f, jnp.int32)
                    va = plsc.load_gather(ra[_k2], [ridx, cidx])
                    vb = plsc.load_gather(rb[_k2], [ridx, cidx])
                    acc = acc + va * vb
                acc_v[_k2][pl.ds(g * 16, 16)] = acc
                return carry2

            lax.fori_loop(0, b // 16, grp, 0)
            dsc_w[j] = pltpu.async_copy(acc_v[k2], out_hbm.at[pl.ds(base + j * b, b)], smw[k2])
        for j in range(max(0, n - 2), n):
            dsc_w[j].wait()

    return k


def _dis_tc(degp):
    def body(d_ref, o_ref):
        o_ref[...] = lax.rsqrt(1.0 + d_ref[0, :] + d_ref[1, :])

    return pl.pallas_call(
        body,
        out_shape=jax.ShapeDtypeStruct((NPAD,), jnp.float32),
    )(degp)


def _mm1(x, w, dis_col, bm):
    m, kdim = x.shape
    nn = w.shape[1]

    def body(x_ref, w_ref, d_ref, o_ref):
        p = jnp.dot(x_ref[...], w_ref[...], preferred_element_type=jnp.float32)
        o_ref[...] = p * d_ref[...]

    return pl.pallas_call(
        body,
        grid=(m // bm,),
        in_specs=[
            pl.BlockSpec((bm, kdim), lambda i: (i, 0)),
            pl.BlockSpec((kdim, nn), lambda i: (0, 0)),
            pl.BlockSpec((bm, 1), lambda i: (i, 0)),
        ],
        out_specs=pl.BlockSpec((bm, nn), lambda i: (i, 0)),
        out_shape=jax.ShapeDtypeStruct((m, nn), jnp.float32),
    )(x, w, dis_col)


def _scale_tc(p, dis_col, bm):
    m, d = p.shape

    def body(p_ref, d_ref, o_ref):
        o_ref[...] = p_ref[...] * d_ref[...]

    return pl.pallas_call(
        body,
        grid=(m // bm,),
        in_specs=[
            pl.BlockSpec((bm, d), lambda i: (i, 0)),
            pl.BlockSpec((bm, 1), lambda i: (i, 0)),
        ],
        out_specs=pl.BlockSpec((bm, d), lambda i: (i, 0)),
        out_shape=jax.ShapeDtypeStruct((m, d), jnp.float32),
    )(p, dis_col)


def _mm2(s1p, t1, dis_col, b1, w2, bm):
    m = t1.shape[0]
    d_in = t1.shape[1]
    d_out = w2.shape[1]

    def body(a_ref, b_ref, t_ref, d_ref, bias_ref, w_ref, o_ref):
        s = t_ref[...] + a_ref[0] + b_ref[0]
        h = jnp.maximum(s * d_ref[...] + bias_ref[...], 0.0)
        o_ref[...] = jnp.dot(h, w_ref[...], preferred_element_type=jnp.float32) * d_ref[...]

    return pl.pallas_call(
        body,
        grid=(m // bm,),
        in_specs=[
            pl.BlockSpec((1, bm, d_in), lambda i: (0, i, 0)),
            pl.BlockSpec((1, bm, d_in), lambda i: (1, i, 0)),
            pl.BlockSpec((bm, d_in), lambda i: (i, 0)),
            pl.BlockSpec((bm, 1), lambda i: (i, 0)),
            pl.BlockSpec((d_in,), lambda i: (0,)),
            pl.BlockSpec((d_in, d_out), lambda i: (0, 0)),
        ],
        out_specs=pl.BlockSpec((bm, d_out), lambda i: (i, 0)),
        out_shape=jax.ShapeDtypeStruct((m, d_out), jnp.float32),
    )(s1p, s1p, t1, dis_col, b1, w2)


def _h2_tc(s2p, t2, dis_col, b2, bm):
    m, d = t2.shape

    def body(a_ref, b_ref, t_ref, d_ref, bias_ref, o_ref):
        s = t_ref[...] + a_ref[0] + b_ref[0]
        o_ref[...] = s * d_ref[...] + bias_ref[...]

    return pl.pallas_call(
        body,
        grid=(m // bm,),
        in_specs=[
            pl.BlockSpec((1, bm, d), lambda i: (0, i, 0)),
            pl.BlockSpec((1, bm, d), lambda i: (1, i, 0)),
            pl.BlockSpec((bm, d), lambda i: (i, 0)),
            pl.BlockSpec((bm, 1), lambda i: (i, 0)),
            pl.BlockSpec((d,), lambda i: (0,)),
        ],
        out_specs=pl.BlockSpec((bm, d), lambda i: (i, 0)),
        out_shape=jax.ShapeDtypeStruct((m, d), jnp.float32),
    )(s2p, s2p, t2, dis_col, b2)


def kernel(pos_edge_index, neg_edge_index, x, train_pos_edge_index, W1, b1, W2, b2):
    e = train_pos_edge_index.shape[1]
    src = train_pos_edge_index[0]
    dst = train_pos_edge_index[1]

    b_deg = 2000
    degp = _sc_degree(e, b_deg)(
        dst,
        jnp.zeros((NPAD, 1), jnp.float32),
        jnp.ones((b_deg, 1), jnp.float32),
    )
    dis = _dis_tc(degp.reshape(NC, NPAD))
    dis_col = dis.reshape(NPAD, 1)[:N]

    t1 = _mm1(x, W1, dis_col, 400)
    s1p = _sc_scatter_rows(32, e, 1000)(t1, src, dst, jnp.zeros((NPAD, 32), jnp.float32))
    t2 = _mm2(s1p[:, :N], t1, dis_col, b1, W2, 400)
    s2p = _sc_scatter_rows(16, e, 1000)(t2, src, dst, jnp.zeros((NPAD, 16), jnp.float32))
    h2 = _h2_tc(s2p[:, :N], t2, dis_col, b2, 1000)

    n_sc = pos_edge_index.shape[1] + neg_edge_index.shape[1]
    e_pad = 204800
    pad = jnp.zeros((e_pad - n_sc,), pos_edge_index.dtype)
    ii = jnp.concatenate([pos_edge_index[1], neg_edge_index[1], pad])
    jj = jnp.concatenate([pos_edge_index[0], neg_edge_index[0], pad])
    return _sc_edge_dot(e_pad, 1600)(h2, ii, jj)[:n_sc]


# speedup vs baseline: 1.0710x; 1.0710x over previous
import functools

import jax
import jax.numpy as jnp
from jax import lax
from jax.experimental import pallas as pl
from jax.experimental.pallas import tpu as pltpu
from jax.experimental.pallas import tpu_sc as plsc

NC = 2
NS = 16
N = 10000
NPAD = 10240


def _mesh():
    return plsc.VectorSubcoreMesh(
        core_axis_name="c", subcore_axis_name="s", num_cores=NC, num_subcores=NS
    )


_SC_PARAMS = pltpu.CompilerParams(use_tc_tiling_on_sc=False, needs_layout_passes=False)


def _sc_degree(e, b):
    et = e // (NC * NS)
    rt = NPAD // NS

    n = et // b

    @functools.partial(
        pl.kernel,
        out_type=jax.ShapeDtypeStruct((NC, NPAD, 1), jnp.float32),
        mesh=_mesh(),
        compiler_params=_SC_PARAMS,
        scratch_types=[
            pltpu.VMEM((b,), jnp.int32),
            pltpu.VMEM((b,), jnp.int32),
            pltpu.VMEM((b, 1), jnp.float32),
            pltpu.VMEM_SHARED((NPAD, 1), jnp.float32),
            pltpu.SemaphoreType.DMA,
            pltpu.SemaphoreType.DMA,
        ],
    )
    def k(dst_hbm, zeros_hbm, ones_hbm, out_hbm, di0, di1, ones_v, acc_sh, smi0, smi1):
        didx = [di0, di1]
        smi = [smi0, smi1]
        c = lax.axis_index("c")
        s = lax.axis_index("s")
        rbase = s * rt
        ebase = (c * NS + s) * et
        dsc = [None] * n

        def issue_idx(j):
            k2 = j % 2
            dsc[j] = pltpu.async_copy(dst_hbm.at[pl.ds(ebase + j * b, b)], didx[k2], smi[k2])

        issue_idx(0)
        if n > 1:
            issue_idx(1)
        pltpu.sync_copy(zeros_hbm.at[pl.ds(rbase, rt)], acc_sh.at[pl.ds(rbase, rt)])
        pltpu.sync_copy(ones_hbm, ones_v)
        plsc.subcore_barrier()
        for j in range(n):
            dsc[j].wait()
            pltpu.sync_copy(ones_v, acc_sh.at[didx[j % 2]], add=True)
            if j + 2 < n:
                issue_idx(j + 2)
        plsc.subcore_barrier()
        pltpu.sync_copy(acc_sh.at[pl.ds(rbase, rt)], out_hbm.at[c].at[pl.ds(rbase, rt)])

    return k


def _sc_scatter_rows(d, e, b):
    ec = e // NC
    et = ec // NS
    rt = NPAD // NS

    n = et // b

    @functools.partial(
        pl.kernel,
        out_type=jax.ShapeDtypeStruct((NC, NPAD, d), jnp.float32),
        mesh=_mesh(),
        compiler_params=_SC_PARAMS,
        scratch_types=[
            pltpu.VMEM((b,), jnp.int32),
            pltpu.VMEM((b,), jnp.int32),
            pltpu.VMEM((b,), jnp.int32),
            pltpu.VMEM((b,), jnp.int32),
            pltpu.VMEM((b, d), jnp.float32),
            pltpu.VMEM((b, d), jnp.float32),
            pltpu.VMEM_SHARED((NPAD, d), jnp.float32),
            pltpu.SemaphoreType.DMA,
            pltpu.SemaphoreType.DMA,
            pltpu.SemaphoreType.DMA,
            pltpu.SemaphoreType.DMA,
        ],
    )
    def k(t_hbm, src_hbm, dst_hbm, zeros_hbm, out_hbm,
          si0, si1, di0, di1, r0, r1, acc_sh, smi0, smi1, smr0, smr1):
        sidx = [si0, si1]
        didx = [di0, di1]
        rows = [r0, r1]
        smi = [smi0, smi1]
        smr = [smr0, smr1]
        c = lax.axis_index("c")
        s = lax.axis_index("s")
        rbase = s * rt
        ebase = c * ec + s * et
        dsc_i = [None] * n
        dsc_d = [None] * n
        dsc_r = [None] * n

        def issue_idx(j):
            k2 = j % 2
            off = ebase + j * b
            dsc_i[j] = pltpu.async_copy(src_hbm.at[pl.ds(off, b)], sidx[k2], smi[k2])
            dsc_d[j] = pltpu.async_copy(dst_hbm.at[pl.ds(off, b)], didx[k2], smi[k2])

        def issue_gather(j):
            k2 = j % 2
            dsc_i[j].wait()
            dsc_d[j].wait()
            dsc_r[j] = pltpu.async_copy(t_hbm.at[sidx[k2]], rows[k2], smr[k2])

        issue_idx(0)
        issue_gather(0)
        if n > 1:
            issue_idx(1)
        pltpu.sync_copy(zeros_hbm.at[pl.ds(rbase, rt)], acc_sh.at[pl.ds(rbase, rt)])
        plsc.subcore_barrier()
        for j in range(n):
            if j + 1 < n:
                issue_gather(j + 1)
            dsc_r[j].wait()
            pltpu.sync_copy(rows[j % 2], acc_sh.at[didx[j % 2]], add=True)
            if j + 2 < n:
                issue_idx(j + 2)
        plsc.subcore_barrier()
        pltpu.sync_copy(acc_sh.at[pl.ds(rbase, rt)], out_hbm.at[c].at[pl.ds(rbase, rt)])

    return k


def _sc_edge_dot(e_pad, b):
    et = e_pad // (NC * NS)

    n = et // b

    @functools.partial(
        pl.kernel,
        out_type=jax.ShapeDtypeStruct((e_pad,), jnp.float32),
        mesh=_mesh(),
        compiler_params=_SC_PARAMS,
        scratch_types=[
            pltpu.VMEM((b,), jnp.int32),
            pltpu.VMEM((b,), jnp.int32),
            pltpu.VMEM((b,), jnp.int32),
            pltpu.VMEM((b,), jnp.int32),
            pltpu.VMEM((b, 16), jnp.float32),
            pltpu.VMEM((b, 16), jnp.float32),
            pltpu.VMEM((b, 16), jnp.float32),
            pltpu.VMEM((b, 16), jnp.float32),
            pltpu.VMEM((b,), jnp.float32),
            pltpu.VMEM((b,), jnp.float32),
            pltpu.SemaphoreType.DMA,
            pltpu.SemaphoreType.DMA,
            pltpu.SemaphoreType.DMA,
            pltpu.SemaphoreType.DMA,
            pltpu.SemaphoreType.DMA,
            pltpu.SemaphoreType.DMA,
        ],
    )
    def k(h_hbm, ii_hbm, jj_hbm, out_hbm,
          ia0, ia1, ib0, ib1, ra0, ra1, rb0, rb1, ac0, ac1,
          smi0, smi1, smr0, smr1, smw0, smw1):
        ia = [ia0, ia1]
        ib = [ib0, ib1]
        ra = [ra0, ra1]
        rb = [rb0, rb1]
        acc_v = [ac0, ac1]
        smi = [smi0, smi1]
        smr = [smr0, smr1]
        smw = [smw0, smw1]
        wid = lax.axis_index("c") * NS + lax.axis_index("s")
        base = wid * et
        lanes = lax.iota(jnp.int32, 16)
        dsc_ia = [None] * n
        dsc_ib = [None] * n
        dsc_ra = [None] * n
        dsc_rb = [None] * n
        dsc_w = [None] * n

        def issue_idx(j):
            k2 = j % 2
            off = base + j * b
            dsc_ia[j] = pltpu.async_copy(ii_hbm.at[pl.ds(off, b)], ia[k2], smi[k2])
            dsc_ib[j] = pltpu.async_copy(jj_hbm.at[pl.ds(off, b)], ib[k2], smi[k2])

        def issue_gather(j):
            k2 = j % 2
            dsc_ia[j].wait()
            dsc_ib[j].wait()
            dsc_ra[j] = pltpu.async_copy(h_hbm.at[ia[k2]], ra[k2], smr[k2])
            dsc_rb[j] = pltpu.async_copy(h_hbm.at[ib[k2]], rb[k2], smr[k2])

        issue_idx(0)
        issue_gather(0)
        if n > 1:
            issue_idx(1)
        for j in range(n):
            k2 = j % 2
            if j + 1 < n:
                issue_gather(j + 1)
            if j >= 2:
                dsc_w[j - 2].wait()
            dsc_ra[j].wait()
            dsc_rb[j].wait()
            if j + 2 < n:
                issue_idx(j + 2)

            def grp(g, carry2, _k2=k2):
                ridx = g * 16 + lanes
                acc = jnp.zeros((16,), jnp.float32)
                for f in range(16):
                    cidx = jnp.full((16,), f, jnp.int32)
                    va = plsc.load_gather(ra[_k2], [ridx, cidx])
                    vb = plsc.load_gather(rb[_k2], [ridx, cidx])
                    acc = acc + va * vb
                acc_v[_k2][pl.ds(g * 16, 16)] = acc
                return carry2

            lax.fori_loop(0, b // 16, grp, 0)
            dsc_w[j] = pltpu.async_copy(acc_v[k2], out_hbm.at[pl.ds(base + j * b, b)], smw[k2])
        for j in range(max(0, n - 2), n):
            dsc_w[j].wait()

    return k


def _mm1(x, w, degp, bm):
    m, kdim = x.shape
    nn = w.shape[1]

    def body(x_ref, w_ref, da_ref, db_ref, o_ref):
        dis = lax.rsqrt(1.0 + da_ref[0] + db_ref[0])
        p = jnp.dot(x_ref[...], w_ref[...], preferred_element_type=jnp.float32)
        o_ref[...] = p * dis

    return pl.pallas_call(
        body,
        grid=(m // bm,),
        in_specs=[
            pl.BlockSpec((bm, kdim), lambda i: (i, 0)),
            pl.BlockSpec((kdim, nn), lambda i: (0, 0)),
            pl.BlockSpec((1, bm, 1), lambda i: (0, i, 0)),
            pl.BlockSpec((1, bm, 1), lambda i: (1, i, 0)),
        ],
        out_specs=pl.BlockSpec((bm, nn), lambda i: (i, 0)),
        out_shape=jax.ShapeDtypeStruct((m, nn), jnp.float32),
    )(x, w, degp, degp)


def _mm2(s1p, t1, degp, b1, w2, bm):
    m = t1.shape[0]
    d_in = t1.shape[1]
    d_out = w2.shape[1]

    def body(a_ref, b_ref, t_ref, da_ref, db_ref, bias_ref, w_ref, o_ref):
        dis = lax.rsqrt(1.0 + da_ref[0] + db_ref[0])
        s = t_ref[...] + a_ref[0] + b_ref[0]
        h = jnp.maximum(s * dis + bias_ref[...], 0.0)
        o_ref[...] = jnp.dot(h, w_ref[...], preferred_element_type=jnp.float32) * dis

    return pl.pallas_call(
        body,
        grid=(m // bm,),
        in_specs=[
            pl.BlockSpec((1, bm, d_in), lambda i: (0, i, 0)),
            pl.BlockSpec((1, bm, d_in), lambda i: (1, i, 0)),
            pl.BlockSpec((bm, d_in), lambda i: (i, 0)),
            pl.BlockSpec((1, bm, 1), lambda i: (0, i, 0)),
            pl.BlockSpec((1, bm, 1), lambda i: (1, i, 0)),
            pl.BlockSpec((d_in,), lambda i: (0,)),
            pl.BlockSpec((d_in, d_out), lambda i: (0, 0)),
        ],
        out_specs=pl.BlockSpec((bm, d_out), lambda i: (i, 0)),
        out_shape=jax.ShapeDtypeStruct((m, d_out), jnp.float32),
    )(s1p, s1p, t1, degp, degp, b1, w2)


def _h2_tc(s2p, t2, degp, b2, bm):
    m, d = t2.shape

    def body(a_ref, b_ref, t_ref, da_ref, db_ref, bias_ref, o_ref):
        dis = lax.rsqrt(1.0 + da_ref[0] + db_ref[0])
        s = t_ref[...] + a_ref[0] + b_ref[0]
        o_ref[...] = s * dis + bias_ref[...]

    return pl.pallas_call(
        body,
        grid=(m // bm,),
        in_specs=[
            pl.BlockSpec((1, bm, d), lambda i: (0, i, 0)),
            pl.BlockSpec((1, bm, d), lambda i: (1, i, 0)),
            pl.BlockSpec((bm, d), lambda i: (i, 0)),
            pl.BlockSpec((1, bm, 1), lambda i: (0, i, 0)),
            pl.BlockSpec((1, bm, 1), lambda i: (1, i, 0)),
            pl.BlockSpec((d,), lambda i: (0,)),
        ],
        out_specs=pl.BlockSpec((bm, d), lambda i: (i, 0)),
        out_shape=jax.ShapeDtypeStruct((m, d), jnp.float32),
    )(s2p, s2p, t2, degp, degp, b2)


def kernel(pos_edge_index, neg_edge_index, x, train_pos_edge_index, W1, b1, W2, b2):
    e = train_pos_edge_index.shape[1]
    src = train_pos_edge_index[0]
    dst = train_pos_edge_index[1]

    b_deg = 2000
    degp = _sc_degree(e, b_deg)(
        dst,
        jnp.zeros((NPAD, 1), jnp.float32),
        jnp.ones((b_deg, 1), jnp.float32),
    )
    t1 = _mm1(x, W1, degp, 400)
    s1p = _sc_scatter_rows(32, e, 1000)(t1, src, dst, jnp.zeros((NPAD, 32), jnp.float32))
    t2 = _mm2(s1p, t1, degp, b1, W2, 400)
    s2p = _sc_scatter_rows(16, e, 1000)(t2, src, dst, jnp.zeros((NPAD, 16), jnp.float32))
    h2 = _h2_tc(s2p, t2, degp, b2, 1000)

    n_sc = pos_edge_index.shape[1] + neg_edge_index.shape[1]
    e_pad = 204800
    pad = jnp.zeros((e_pad - n_sc,), pos_edge_index.dtype)
    ii = jnp.concatenate([pos_edge_index[1], neg_edge_index[1], pad])
    jj = jnp.concatenate([pos_edge_index[0], neg_edge_index[0], pad])
    return _sc_edge_dot(e_pad, 1600)(h2, ii, jj)[:n_sc]


# speedup vs baseline: 1.1937x; 1.1146x over previous
import functools

import jax
import jax.numpy as jnp
from jax import lax
from jax.experimental import pallas as pl
from jax.experimental.pallas import tpu as pltpu
from jax.experimental.pallas import tpu_sc as plsc

NC = 2
NS = 16
N = 10000
NPAD = 10240


def _mesh():
    return plsc.VectorSubcoreMesh(
        core_axis_name="c", subcore_axis_name="s", num_cores=NC, num_subcores=NS
    )


_SC_PARAMS = pltpu.CompilerParams(use_tc_tiling_on_sc=False, needs_layout_passes=False)


def _sc_degree(e, b):
    et = e // (NC * NS)
    rt = NPAD // NS

    n = et // b

    @functools.partial(
        pl.kernel,
        out_type=jax.ShapeDtypeStruct((NC, NPAD, 1), jnp.float32),
        mesh=_mesh(),
        compiler_params=_SC_PARAMS,
        scratch_types=[
            pltpu.VMEM((b,), jnp.int32),
            pltpu.VMEM((b,), jnp.int32),
            pltpu.VMEM((b, 1), jnp.float32),
            pltpu.VMEM_SHARED((NPAD, 1), jnp.float32),
            pltpu.SemaphoreType.DMA,
            pltpu.SemaphoreType.DMA,
        ],
    )
    def k(dst_hbm, zeros_hbm, ones_hbm, out_hbm, di0, di1, ones_v, acc_sh, smi0, smi1):
        didx = [di0, di1]
        smi = [smi0, smi1]
        c = lax.axis_index("c")
        s = lax.axis_index("s")
        rbase = s * rt
        ebase = (c * NS + s) * et
        dsc = [None] * n

        def issue_idx(j):
            k2 = j % 2
            dsc[j] = pltpu.async_copy(dst_hbm.at[pl.ds(ebase + j * b, b)], didx[k2], smi[k2])

        issue_idx(0)
        if n > 1:
            issue_idx(1)
        pltpu.sync_copy(zeros_hbm.at[pl.ds(rbase, rt)], acc_sh.at[pl.ds(rbase, rt)])
        pltpu.sync_copy(ones_hbm, ones_v)
        plsc.subcore_barrier()
        for j in range(n):
            dsc[j].wait()
            pltpu.sync_copy(ones_v, acc_sh.at[didx[j % 2]], add=True)
            if j + 2 < n:
                issue_idx(j + 2)
        plsc.subcore_barrier()
        pltpu.sync_copy(acc_sh.at[pl.ds(rbase, rt)], out_hbm.at[c].at[pl.ds(rbase, rt)])

    return k


def _sc_scatter_rows(d, e, b):
    ec = e // NC
    et = ec // NS
    rt = NPAD // NS

    n = et // b

    @functools.partial(
        pl.kernel,
        out_type=jax.ShapeDtypeStruct((NC, NPAD, d), jnp.float32),
        mesh=_mesh(),
        compiler_params=_SC_PARAMS,
        scratch_types=[
            pltpu.VMEM((b,), jnp.int32),
            pltpu.VMEM((b,), jnp.int32),
            pltpu.VMEM((b,), jnp.int32),
            pltpu.VMEM((b,), jnp.int32),
            pltpu.VMEM((b, d), jnp.float32),
            pltpu.VMEM((b, d), jnp.float32),
            pltpu.VMEM_SHARED((NPAD, d), jnp.float32),
            pltpu.SemaphoreType.DMA,
            pltpu.SemaphoreType.DMA,
            pltpu.SemaphoreType.DMA,
            pltpu.SemaphoreType.DMA,
        ],
    )
    def k(t_hbm, src_hbm, dst_hbm, zeros_hbm, out_hbm,
          si0, si1, di0, di1, r0, r1, acc_sh, smi0, smi1, smr0, smr1):
        sidx = [si0, si1]
        didx = [di0, di1]
        rows = [r0, r1]
        smi = [smi0, smi1]
        smr = [smr0, smr1]
        c = lax.axis_index("c")
        s = lax.axis_index("s")
        rbase = s * rt
        ebase = c * ec + s * et
        dsc_i = [None] * n
        dsc_d = [None] * n
        dsc_r = [None] * n

        def issue_idx(j):
            k2 = j % 2
            off = ebase + j * b
            dsc_i[j] = pltpu.async_copy(src_hbm.at[pl.ds(off, b)], sidx[k2], smi[k2])
            dsc_d[j] = pltpu.async_copy(dst_hbm.at[pl.ds(off, b)], didx[k2], smi[k2])

        def issue_gather(j):
            k2 = j % 2
            dsc_i[j].wait()
            dsc_d[j].wait()
            dsc_r[j] = pltpu.async_copy(t_hbm.at[sidx[k2]], rows[k2], smr[k2])

        issue_idx(0)
        issue_gather(0)
        if n > 1:
            issue_idx(1)
        pltpu.sync_copy(zeros_hbm.at[pl.ds(rbase, rt)], acc_sh.at[pl.ds(rbase, rt)])
        plsc.subcore_barrier()
        for j in range(n):
            if j + 1 < n:
                issue_gather(j + 1)
            dsc_r[j].wait()
            pltpu.sync_copy(rows[j % 2], acc_sh.at[didx[j % 2]], add=True)
            if j + 2 < n:
                issue_idx(j + 2)
        plsc.subcore_barrier()
        pltpu.sync_copy(acc_sh.at[pl.ds(rbase, rt)], out_hbm.at[c].at[pl.ds(rbase, rt)])

    return k


def _sc_edge_dot(e_pad, b):
    et = e_pad // (NC * NS)

    n = et // b

    @functools.partial(
        pl.kernel,
        out_type=jax.ShapeDtypeStruct((e_pad,), jnp.float32),
        mesh=_mesh(),
        compiler_params=_SC_PARAMS,
        scratch_types=[
            pltpu.VMEM((b,), jnp.int32),
            pltpu.VMEM((b,), jnp.int32),
            pltpu.VMEM((b,), jnp.int32),
            pltpu.VMEM((b,), jnp.int32),
            pltpu.VMEM((b, 16), jnp.float32),
            pltpu.VMEM((b, 16), jnp.float32),
            pltpu.VMEM((b, 16), jnp.float32),
            pltpu.VMEM((b, 16), jnp.float32),
            pltpu.VMEM((b,), jnp.float32),
            pltpu.VMEM((b,), jnp.float32),
            pltpu.VMEM_SHARED((N, 16), jnp.float32),
            pltpu.SemaphoreType.DMA,
            pltpu.SemaphoreType.DMA,
            pltpu.SemaphoreType.DMA,
            pltpu.SemaphoreType.DMA,
            pltpu.SemaphoreType.DMA,
            pltpu.SemaphoreType.DMA,
        ],
    )
    def k(h_hbm, ii_hbm, jj_hbm, out_hbm,
          ia0, ia1, ib0, ib1, ra0, ra1, rb0, rb1, ac0, ac1, h_sh,
          smi0, smi1, smr0, smr1, smw0, smw1):
        ia = [ia0, ia1]
        ib = [ib0, ib1]
        ra = [ra0, ra1]
        rb = [rb0, rb1]
        acc_v = [ac0, ac1]
        smi = [smi0, smi1]
        smr = [smr0, smr1]
        smw = [smw0, smw1]
        wid = lax.axis_index("c") * NS + lax.axis_index("s")
        base = wid * et
        lanes = lax.iota(jnp.int32, 16)
        dsc_ia = [None] * n
        dsc_ib = [None] * n
        dsc_ra = [None] * n
        dsc_rb = [None] * n
        dsc_w = [None] * n

        def issue_idx(j):
            k2 = j % 2
            off = base + j * b
            dsc_ia[j] = pltpu.async_copy(ii_hbm.at[pl.ds(off, b)], ia[k2], smi[k2])
            dsc_ib[j] = pltpu.async_copy(jj_hbm.at[pl.ds(off, b)], ib[k2], smi[k2])

        def issue_gather(j):
            k2 = j % 2
            dsc_ia[j].wait()
            dsc_ib[j].wait()
            dsc_ra[j] = pltpu.async_copy(h_sh.at[ia[k2]], ra[k2], smr[k2])
            dsc_rb[j] = pltpu.async_copy(h_sh.at[ib[k2]], rb[k2], smr[k2])

        issue_idx(0)
        if n > 1:
            issue_idx(1)
        s = lax.axis_index("s")
        hrt = N // NS
        pltpu.sync_copy(h_hbm.at[pl.ds(s * hrt, hrt)], h_sh.at[pl.ds(s * hrt, hrt)])
        plsc.subcore_barrier()
        issue_gather(0)
        for j in range(n):
            k2 = j % 2
            if j + 1 < n:
                issue_gather(j + 1)
            if j >= 2:
                dsc_w[j - 2].wait()
            dsc_ra[j].wait()
            dsc_rb[j].wait()
            if j + 2 < n:
                issue_idx(j + 2)

            def grp(g, carry2, _k2=k2):
                ridx = g * 16 + lanes
                acc = jnp.zeros((16,), jnp.float32)
                for f in range(16):
                    cidx = jnp.full((16,), f, jnp.int32)
                    va = plsc.load_gather(ra[_k2], [ridx, cidx])
                    vb = plsc.load_gather(rb[_k2], [ridx, cidx])
                    acc = acc + va * vb
                acc_v[_k2][pl.ds(g * 16, 16)] = acc
                return carry2

            lax.fori_loop(0, b // 16, grp, 0)
            dsc_w[j] = pltpu.async_copy(acc_v[k2], out_hbm.at[pl.ds(base + j * b, b)], smw[k2])
        for j in range(max(0, n - 2), n):
            dsc_w[j].wait()

    return k


def _mm1(x, w, degp, bm):
    m, kdim = x.shape
    nn = w.shape[1]

    def body(x_ref, w_ref, da_ref, db_ref, o_ref):
        dis = lax.rsqrt(1.0 + da_ref[0] + db_ref[0])
        p = jnp.dot(x_ref[...], w_ref[...], preferred_element_type=jnp.float32)
        o_ref[...] = p * dis

    return pl.pallas_call(
        body,
        grid=(m // bm,),
        in_specs=[
            pl.BlockSpec((bm, kdim), lambda i: (i, 0)),
            pl.BlockSpec((kdim, nn), lambda i: (0, 0)),
            pl.BlockSpec((1, bm, 1), lambda i: (0, i, 0)),
            pl.BlockSpec((1, bm, 1), lambda i: (1, i, 0)),
        ],
        out_specs=pl.BlockSpec((bm, nn), lambda i: (i, 0)),
        out_shape=jax.ShapeDtypeStruct((m, nn), jnp.float32),
    )(x, w, degp, degp)


def _mm2(s1p, t1, degp, b1, w2, bm):
    m = t1.shape[0]
    d_in = t1.shape[1]
    d_out = w2.shape[1]

    def body(a_ref, b_ref, t_ref, da_ref, db_ref, bias_ref, w_ref, o_ref):
        dis = lax.rsqrt(1.0 + da_ref[0] + db_ref[0])
        s = t_ref[...] + a_ref[0] + b_ref[0]
        h = jnp.maximum(s * dis + bias_ref[...], 0.0)
        o_ref[...] = jnp.dot(h, w_ref[...], preferred_element_type=jnp.float32) * dis

    return pl.pallas_call(
        body,
        grid=(m // bm,),
        in_specs=[
            pl.BlockSpec((1, bm, d_in), lambda i: (0, i, 0)),
            pl.BlockSpec((1, bm, d_in), lambda i: (1, i, 0)),
            pl.BlockSpec((bm, d_in), lambda i: (i, 0)),
            pl.BlockSpec((1, bm, 1), lambda i: (0, i, 0)),
            pl.BlockSpec((1, bm, 1), lambda i: (1, i, 0)),
            pl.BlockSpec((d_in,), lambda i: (0,)),
            pl.BlockSpec((d_in, d_out), lambda i: (0, 0)),
        ],
        out_specs=pl.BlockSpec((bm, d_out), lambda i: (i, 0)),
        out_shape=jax.ShapeDtypeStruct((m, d_out), jnp.float32),
    )(s1p, s1p, t1, degp, degp, b1, w2)


def _h2_tc(s2p, t2, degp, b2, bm):
    m, d = t2.shape

    def body(a_ref, b_ref, t_ref, da_ref, db_ref, bias_ref, o_ref):
        dis = lax.rsqrt(1.0 + da_ref[0] + db_ref[0])
        s = t_ref[...] + a_ref[0] + b_ref[0]
        o_ref[...] = s * dis + bias_ref[...]

    return pl.pallas_call(
        body,
        grid=(m // bm,),
        in_specs=[
            pl.BlockSpec((1, bm, d), lambda i: (0, i, 0)),
            pl.BlockSpec((1, bm, d), lambda i: (1, i, 0)),
            pl.BlockSpec((bm, d), lambda i: (i, 0)),
            pl.BlockSpec((1, bm, 1), lambda i: (0, i, 0)),
            pl.BlockSpec((1, bm, 1), lambda i: (1, i, 0)),
            pl.BlockSpec((d,), lambda i: (0,)),
        ],
        out_specs=pl.BlockSpec((bm, d), lambda i: (i, 0)),
        out_shape=jax.ShapeDtypeStruct((m, d), jnp.float32),
    )(s2p, s2p, t2, degp, degp, b2)


def kernel(pos_edge_index, neg_edge_index, x, train_pos_edge_index, W1, b1, W2, b2):
    e = train_pos_edge_index.shape[1]
    src = train_pos_edge_index[0]
    dst = train_pos_edge_index[1]

    b_deg = 2000
    degp = _sc_degree(e, b_deg)(
        dst,
        jnp.zeros((NPAD, 1), jnp.float32),
        jnp.ones((b_deg, 1), jnp.float32),
    )
    t1 = _mm1(x, W1, degp, 400)
    s1p = _sc_scatter_rows(32, e, 1000)(t1, src, dst, jnp.zeros((NPAD, 32), jnp.float32))
    t2 = _mm2(s1p, t1, degp, b1, W2, 400)
    s2p = _sc_scatter_rows(16, e, 1000)(t2, src, dst, jnp.zeros((NPAD, 16), jnp.float32))
    h2 = _h2_tc(s2p, t2, degp, b2, 1000)

    n_sc = pos_edge_index.shape[1] + neg_edge_index.shape[1]
    e_pad = 204800
    pad = jnp.zeros((e_pad - n_sc,), pos_edge_index.dtype)
    ii = jnp.concatenate([pos_edge_index[1], neg_edge_index[1], pad])
    jj = jnp.concatenate([pos_edge_index[0], neg_edge_index[0], pad])
    return _sc_edge_dot(e_pad, 1600)(h2, ii, jj)[:n_sc]
